# baseline (device time: 34958 ns/iter reference)
import functools

import jax
import jax.numpy as jnp
from jax import lax
from jax.experimental import pallas as pl
from jax.experimental.pallas import tpu as pltpu

N_DEV = 4
N_LAYERS = 3

SEND_ORDER = (2, 1, 3)
WAIT_ORDER = (1, 3, 2)


def kernel(x, Win0, Wout0, Win1, Wout1, Win2, Wout2):
    b, d_shard = x.shape
    h_dim = Win0.shape[1]
    bq = b // N_DEV

    def body(x_hbm, win0h, wout0h, win1h, wout1h, win2h, wout2h, out_ref,
             qsrc0, qsrcN, rs_recv, ag_src, ag_recv, pbuf,
             xv, wv, wo, load_sems,
             rs_ssem, rs_rsem, ag_ssem, ag_rsem):
        my = lax.axis_index("i")

        loads = []
        for i, (src, dst) in enumerate([
                (x_hbm, xv), (win0h, wv.at[0]),
                (win1h, wv.at[1]), (win2h, wv.at[2]),
                (wout0h, wo.at[0]), (wout1h, wo.at[1]), (wout2h, wo.at[2])]):
            cp = pltpu.make_async_copy(src, dst, load_sems.at[i])
            cp.start()
            loads.append(cp)

        barrier = pltpu.get_barrier_semaphore()
        for k in range(1, N_DEV):
            peer = lax.rem(my + k, N_DEV)
            pl.semaphore_signal(barrier, inc=1, device_id=(peer,),
                                device_id_type=pl.DeviceIdType.MESH)
        pl.semaphore_wait(barrier, N_DEV - 1)

        loads[0].wait()
        loads[1].wait()
        x_cur = xv[...].astype(jnp.bfloat16)
        partial = jnp.dot(x_cur, wv[0].astype(jnp.bfloat16),
                          preferred_element_type=jnp.float32)
        pbuf[...] = partial
        for q in range(N_DEV):
            qsrc0[q] = partial[q * bq:(q + 1) * bq, :].astype(jnp.bfloat16)
        rs0 = [None] * (N_DEV - 1)
        for k in SEND_ORDER:
            peer = lax.rem(my + k, N_DEV)
            r = pltpu.make_async_remote_copy(
                src_ref=qsrc0.at[peer],
                dst_ref=rs_recv.at[0, k - 1],
                send_sem=rs_ssem.at[0, k - 1],
                recv_sem=rs_rsem.at[0, k - 1],
                device_id=(peer,),
                device_id_type=pl.DeviceIdType.MESH,
            )
            r.start()
            rs0[k - 1] = r
        for cp in loads[2:]:
            cp.wait()
        acc = pbuf[pl.ds(my * bq, bq), :]
        for k in WAIT_ORDER:
            rs0[k - 1].wait()
            acc = acc + rs_recv[0, k - 1].astype(jnp.float32)
        relu_q = jnp.maximum(acc, 0.0).astype(jnp.bfloat16)

        ag_src[0] = relu_q
        ag = [None] * (N_DEV - 1)
        for k in SEND_ORDER:
            peer = lax.rem(my + k, N_DEV)
            r = pltpu.make_async_remote_copy(
                src_ref=ag_src.at[0],
                dst_ref=ag_recv.at[0, k - 1],
                send_sem=ag_ssem.at[0, k - 1],
                recv_sem=ag_rsem.at[0, k - 1],
                device_id=(peer,),
                device_id_type=pl.DeviceIdType.MESH,
            )
            r.start()
            ag[k - 1] = r

        for L in range(N_LAYERS):
            last = L == N_LAYERS - 1
            wout = wo[L].astype(jnp.bfloat16)
            if not last:
                win_next = wv[L + 1].astype(jnp.bfloat16)

            res_my = jnp.dot(relu_q, wout,
                             preferred_element_type=jnp.float32)
            if last:
                out_ref[pl.ds(my * bq, bq), :] = res_my
            else:
                acc = jnp.dot(res_my.astype(jnp.bfloat16), win_next,
                              preferred_element_type=jnp.float32)

            rs_next = []
            for k in WAIT_ORDER:
                ag[k - 1].wait()
                p = lax.rem(my - k + N_DEV, N_DEV)
                xq = jnp.dot(ag_recv[L, k - 1], wout,
                             preferred_element_type=jnp.float32)
                if last:
                    out_ref[pl.ds(p * bq, bq), :] = xq
                else:
                    p2 = jnp.dot(xq.astype(jnp.bfloat16), win_next,
                                 preferred_element_type=jnp.float32)
                    qsrcN[L, k - 1] = p2.astype(jnp.bfloat16)
                    r = pltpu.make_async_remote_copy(
                        src_ref=qsrcN.at[L, k - 1],
                        dst_ref=rs_recv.at[L + 1, 3 - k],
                        send_sem=rs_ssem.at[L + 1, 3 - k],
                        recv_sem=rs_rsem.at[L + 1, 3 - k],
                        device_id=(p,),
                        device_id_type=pl.DeviceIdType.MESH,
                    )
                    r.start()
                    rs_next.append((3 - k, r))
            if last:
                break

            for slot, r in rs_next:
                r.wait()
                acc = acc + rs_recv[L + 1, slot].astype(jnp.float32)
            relu_q = jnp.maximum(acc, 0.0).astype(jnp.bfloat16)

            ag_src[L + 1] = relu_q
            ag = [None] * (N_DEV - 1)
            for k in SEND_ORDER:
                peer = lax.rem(my + k, N_DEV)
                r = pltpu.make_async_remote_copy(
                    src_ref=ag_src.at[L + 1],
                    dst_ref=ag_recv.at[L + 1, k - 1],
                    send_sem=ag_ssem.at[L + 1, k - 1],
                    recv_sem=ag_rsem.at[L + 1, k - 1],
                    device_id=(peer,),
                    device_id_type=pl.DeviceIdType.MESH,
                )
                r.start()
                ag[k - 1] = r

        @functools.partial(pl.run_scoped, exit_sem=pltpu.SemaphoreType.REGULAR)
        def _(exit_sem):
            for k in range(1, N_DEV):
                peer = lax.rem(my + k, N_DEV)
                pl.semaphore_signal(exit_sem, inc=1, device_id=(peer,),
                                    device_id_type=pl.DeviceIdType.MESH)
            pl.semaphore_wait(exit_sem, N_DEV - 1)

    return pl.pallas_call(
        body,
        out_shape=jax.ShapeDtypeStruct((b, d_shard), jnp.float32),
        in_specs=[pl.BlockSpec(memory_space=pl.ANY)] * 7,
        out_specs=pl.BlockSpec(memory_space=pltpu.VMEM),
        scratch_shapes=[
            pltpu.VMEM((N_DEV, bq, h_dim), jnp.bfloat16),
            pltpu.VMEM((N_LAYERS - 1, N_DEV - 1, bq, h_dim), jnp.bfloat16),
            pltpu.VMEM((N_LAYERS, N_DEV - 1, bq, h_dim), jnp.bfloat16),
            pltpu.VMEM((N_LAYERS, bq, h_dim), jnp.bfloat16),
            pltpu.VMEM((N_LAYERS, N_DEV - 1, bq, h_dim), jnp.bfloat16),
            pltpu.VMEM((b, h_dim), jnp.float32),
            pltpu.VMEM((b, d_shard), jnp.float32),
            pltpu.VMEM((N_LAYERS, d_shard, h_dim), jnp.float32),
            pltpu.VMEM((N_LAYERS, h_dim, d_shard), jnp.float32),
            pltpu.SemaphoreType.DMA((7,)),
            pltpu.SemaphoreType.DMA((N_LAYERS, N_DEV - 1)),
            pltpu.SemaphoreType.DMA((N_LAYERS, N_DEV - 1)),
            pltpu.SemaphoreType.DMA((N_LAYERS, N_DEV - 1)),
            pltpu.SemaphoreType.DMA((N_LAYERS, N_DEV - 1)),
        ],
        compiler_params=pltpu.CompilerParams(collective_id=0),
    )(x, Win0, Wout0, Win1, Wout1, Win2, Wout2)
